# baseline (speedup 1.0000x reference)
import functools

import jax
import jax.numpy as jnp
from jax import lax
from jax.experimental import pallas as pl
from jax.experimental.pallas import tpu as pltpu
from jax.experimental.pallas import tpu_sc as plsc

NUM_FIELDS = 26
VOCAB = 100000
EMBED_DIM = 64
BATCH = 4096
LANES = 16
NBUF = 4
ROWS = BATCH // 32


def _sc_body(tab_ref, xt_ref, out_ref, idx_v, par_v, buf_v, acc_v, sems):
    nc = 2
    wid = lax.axis_index("s") * nc + lax.axis_index("c")
    base = wid * ROWS

    for f in range(NUM_FIELDS):
        pltpu.sync_copy(xt_ref.at[f, pl.ds(base, ROWS)], idx_v.at[f])
        for i in range(ROWS // LANES):
            sl = pl.ds(i * LANES, LANES)
            flat = idx_v[f, sl] + f * VOCAB
            idx_v[f, sl] = lax.shift_right_logical(flat, 1)
            par_v[f, sl] = lax.bitwise_and(flat, 1)

    descs = {}
    for f in range(NBUF):
        descs[f] = pltpu.async_copy(
            tab_ref.at[idx_v.at[f]], buf_v.at[f], sems.at[f])

    for f in range(NUM_FIELDS):
        slot = f % NBUF
        descs[f].wait()
        buf = buf_v.at[slot]

        for g in range(ROWS // LANES):
            rowv = lax.iota(jnp.int32, LANES) + g * LANES
            colbase = par_v[f, pl.ds(g * LANES, LANES)] * EMBED_DIM

            def accum(c, carry, rowv=rowv, colbase=colbase, buf=buf, f=f, g=g):
                val = plsc.load_gather(buf, [rowv, colbase + c])
                dst = acc_v.at[c, pl.ds(g * LANES, LANES)]
                if f == 0:
                    acc_v[c, pl.ds(g * LANES, LANES)] = val
                else:
                    plsc.addupdate(dst, val)
                return carry

            lax.fori_loop(0, EMBED_DIM, accum, 0)

        nxt = f + NBUF
        if nxt < NUM_FIELDS:
            descs[nxt] = pltpu.async_copy(
                tab_ref.at[idx_v.at[nxt]], buf_v.at[slot], sems.at[slot])

    pltpu.sync_copy(acc_v, out_ref.at[:, pl.ds(base, ROWS)])


def kernel(x, tables):
    xt = x.astype(jnp.int32).T
    tab = tables.reshape(NUM_FIELDS * VOCAB // 2, 2 * EMBED_DIM)
    mesh = plsc.VectorSubcoreMesh(core_axis_name="c", subcore_axis_name="s")
    run = functools.partial(
        pl.kernel,
        mesh=mesh,
        out_type=jax.ShapeDtypeStruct((EMBED_DIM, BATCH), jnp.float32),
        scratch_types=[
            pltpu.VMEM((32, ROWS), jnp.int32),
            pltpu.VMEM((32, ROWS), jnp.int32),
            pltpu.VMEM((NBUF, ROWS, 2 * EMBED_DIM), jnp.float32),
            pltpu.VMEM((EMBED_DIM, ROWS), jnp.float32),
            pltpu.SemaphoreType.DMA((NBUF,)),
        ],
        compiler_params=pltpu.CompilerParams(needs_layout_passes=False),
    )(_sc_body)
    return run(tab, xt).T


# speedup vs baseline: 1.1061x; 1.1061x over previous
import functools

import jax
import jax.numpy as jnp
from jax import lax
from jax.experimental import pallas as pl
from jax.experimental.pallas import tpu as pltpu
from jax.experimental.pallas import tpu_sc as plsc

NUM_FIELDS = 26
VOCAB = 100000
EMBED_DIM = 64
BATCH = 4096
LANES = 16
NBUF = 4
ROWS = BATCH // 32


def _sc_body(tab_ref, xt_ref, out_ref, idx_v, buf_v, acc_v, sems):
    nc = 2
    wid = lax.axis_index("s") * nc + lax.axis_index("c")
    base = wid * ROWS

    pltpu.sync_copy(xt_ref.at[:, pl.ds(base, ROWS)], idx_v)

    descs = {}
    for f in range(NBUF):
        descs[f] = pltpu.async_copy(
            tab_ref.at[f].at[idx_v.at[f]], buf_v.at[f], sems.at[f])

    for f in range(NUM_FIELDS):
        slot = f % NBUF
        descs[f].wait()

        def accum(r, carry, slot=slot, f=f):
            for c in range(EMBED_DIM // LANES):
                sl = pl.ds(c * LANES, LANES)
                if f == 0:
                    acc_v[r, sl] = buf_v[slot, r, sl]
                else:
                    plsc.addupdate(acc_v.at[r, sl], buf_v[slot, r, sl])
            return carry

        lax.fori_loop(0, ROWS, accum, 0)

        nxt = f + NBUF
        if nxt < NUM_FIELDS:
            descs[nxt] = pltpu.async_copy(
                tab_ref.at[nxt].at[idx_v.at[nxt]], buf_v.at[slot],
                sems.at[slot])

    pltpu.sync_copy(acc_v, out_ref.at[pl.ds(base, ROWS)])


def kernel(x, tables):
    xt = x.astype(jnp.int32).T
    mesh = plsc.VectorSubcoreMesh(core_axis_name="c", subcore_axis_name="s")
    run = functools.partial(
        pl.kernel,
        mesh=mesh,
        out_type=jax.ShapeDtypeStruct((BATCH, EMBED_DIM), jnp.float32),
        scratch_types=[
            pltpu.VMEM((NUM_FIELDS, ROWS), jnp.int32),
            pltpu.VMEM((NBUF, ROWS, EMBED_DIM), jnp.float32),
            pltpu.VMEM((ROWS, EMBED_DIM), jnp.float32),
            pltpu.SemaphoreType.DMA((NBUF,)),
        ],
        compiler_params=pltpu.CompilerParams(use_tc_tiling_on_sc=False),
    )(_sc_body)
    return run(tables, xt)


# speedup vs baseline: 1.2554x; 1.1350x over previous
import functools

import jax
import jax.numpy as jnp
from jax import lax
from jax.experimental import pallas as pl
from jax.experimental.pallas import tpu as pltpu
from jax.experimental.pallas import tpu_sc as plsc

NUM_FIELDS = 26
VOCAB = 100000
EMBED_DIM = 64
BATCH = 4096
LANES = 16
NBUF = 4
ROWS = BATCH // 32
PADD = 128


def _sc_body(tab_ref, xt_ref, out_ref, idx_v, buf_v, acc_v, sems):
    nc = 2
    wid = lax.axis_index("s") * nc + lax.axis_index("c")
    base = wid * ROWS

    pltpu.sync_copy(xt_ref.at[:, pl.ds(base, ROWS)], idx_v)

    descs = {}
    for f in range(NBUF):
        descs[f] = pltpu.async_copy(
            tab_ref.at[f].at[idx_v.at[f]], buf_v.at[f], sems.at[f])

    for f in range(NUM_FIELDS):
        slot = f % NBUF
        descs[f].wait()

        def accum(r, carry, slot=slot, f=f):
            for c in range(EMBED_DIM // LANES):
                sl = pl.ds(c * LANES, LANES)
                if f == 0:
                    acc_v[r, sl] = buf_v[slot, r, sl]
                else:
                    plsc.addupdate(acc_v.at[r, sl], buf_v[slot, r, sl])
            return carry

        lax.fori_loop(0, ROWS, accum, 0)

        nxt = f + NBUF
        if nxt < NUM_FIELDS:
            descs[nxt] = pltpu.async_copy(
                tab_ref.at[nxt].at[idx_v.at[nxt]], buf_v.at[slot],
                sems.at[slot])

    pltpu.sync_copy(acc_v, out_ref.at[pl.ds(base, ROWS)])


def kernel(x, tables):
    xt = jnp.pad(x.astype(jnp.int32), ((0, 0), (0, 6))).T
    tab = jnp.pad(tables, ((0, 0), (0, 0), (0, PADD - EMBED_DIM)))
    mesh = plsc.VectorSubcoreMesh(core_axis_name="c", subcore_axis_name="s")
    run = functools.partial(
        pl.kernel,
        mesh=mesh,
        out_type=jax.ShapeDtypeStruct((BATCH, PADD), jnp.float32),
        scratch_types=[
            pltpu.VMEM((32, ROWS), jnp.int32),
            pltpu.VMEM((NBUF, ROWS, PADD), jnp.float32),
            pltpu.VMEM((ROWS, PADD), jnp.float32),
            pltpu.SemaphoreType.DMA((NBUF,)),
        ],
        compiler_params=pltpu.CompilerParams(use_tc_tiling_on_sc=True),
    )(_sc_body)
    return run(tab, xt)[:, :EMBED_DIM]
